# baseline (speedup 1.0000x reference)
import functools

import jax
import jax.numpy as jnp
import numpy as np
from jax import lax
from jax.experimental import pallas as pl
from jax.experimental.pallas import tpu as pltpu
from jax.experimental.pallas import tpu_sc as plsc

N = 10000
E = 320000
D = 128
P = 2

NC = 2
NS = 16
NW = NC * NS
F = 144
FE = F - D
B = 128
K = 8
NB = 80
E_PAD = NW * NB * B
N_PAD = 10224
NPT = N_PAD // NS

_sc_mesh = plsc.VectorSubcoreMesh(core_axis_name="c", subcore_axis_name="s")


@functools.partial(
    pl.kernel,
    out_type=(jax.ShapeDtypeStruct((NC, N_PAD, D), jnp.float32),
              jax.ShapeDtypeStruct((NC, N_PAD, D), jnp.float32)),
    mesh=_sc_mesh,
    compiler_params=pltpu.CompilerParams(use_tc_tiling_on_sc=False),
    scratch_types=[
        pltpu.VMEM((K, B), jnp.int32),
        pltpu.VMEM((K, B), jnp.int32),
        pltpu.VMEM((B, D), jnp.float32),
        pltpu.VMEM((B, D), jnp.float32),
        pltpu.VMEM((B, FE), jnp.float32),
        pltpu.VMEM((B, FE), jnp.float32),
        pltpu.VMEM_SHARED((N_PAD, D), jnp.float32),
        pltpu.VMEM_SHARED((N_PAD, FE), jnp.float32),
        pltpu.SemaphoreType.DMA,
        pltpu.SemaphoreType.DMA,
        pltpu.SemaphoreType.DMA,
        pltpu.SemaphoreType.DMA,
    ],
)
def _seg_sum_sc(x_hbm, xe_hbm, edges_hbm, zeros_hbm, outx_hbm,
                oute_hbm, src_v, dst_v, rx0, rx1, re0, re1, accx, acce,
                gsem0, gsem1, ssem0, ssem1):
    cid = lax.axis_index("c")
    sid = lax.axis_index("s")
    wid = cid * NS + sid

    def load_slab(sb):
        pltpu.sync_copy(edges_hbm.at[0, wid, sb], src_v)
        pltpu.sync_copy(edges_hbm.at[1, wid, sb], dst_v)

    def start_gather(k, rx, re, sem):
        pltpu.async_copy(x_hbm.at[dst_v.at[k]], rx, sem)
        pltpu.async_copy(xe_hbm.at[dst_v.at[k]], re, sem)

    def wait_gather(k, rx, re, sem):
        pltpu.make_async_copy(x_hbm.at[dst_v.at[k]], rx, sem).wait()
        pltpu.make_async_copy(xe_hbm.at[dst_v.at[k]], re, sem).wait()

    def start_scatter(k, rx, re, sem):
        pltpu.async_copy(rx, accx.at[src_v.at[k]], sem, add=True)
        pltpu.async_copy(re, acce.at[src_v.at[k]], sem, add=True)

    def wait_scatter(k, rx, re, sem):
        pltpu.make_async_copy(rx, accx.at[src_v.at[k]], sem).wait()
        pltpu.make_async_copy(re, acce.at[src_v.at[k]], sem).wait()

    pltpu.sync_copy(zeros_hbm, accx.at[pl.ds(sid * NPT, NPT)])
    pltpu.sync_copy(zeros_hbm.at[:, pl.ds(0, FE)],
                    acce.at[pl.ds(sid * NPT, NPT)])
    plsc.subcore_barrier()

    def superblock(s, _):
        load_slab(s)
        start_gather(0, rx0, re0, gsem0)
        bufs = [(rx0, re0, gsem0, ssem0), (rx1, re1, gsem1, ssem1)]
        for k in range(K):
            rx, re, gsem, ssem = bufs[k % 2]
            orx, ore, ogsem, ossem = bufs[(k + 1) % 2]
            if k + 1 < K:
                if k >= 1:
                    wait_scatter(k - 1, orx, ore, ossem)
                start_gather(k + 1, orx, ore, ogsem)
            wait_gather(k, rx, re, gsem)
            start_scatter(k, rx, re, ssem)
        wait_scatter(K - 2, rx0, re0, ssem0)
        wait_scatter(K - 1, rx1, re1, ssem1)
        return 0

    lax.fori_loop(0, NB // K, superblock, 0)

    plsc.subcore_barrier()
    r0 = sid * NPT
    pltpu.sync_copy(accx.at[pl.ds(r0, NPT)], outx_hbm.at[cid, pl.ds(r0, NPT)])
    pltpu.sync_copy(acce.at[pl.ds(r0, NPT)],
                    oute_hbm.at[cid, pl.ds(r0, NPT), pl.ds(0, FE)])


def _pre_body(x_ref, w1_ref, bm_ref, wa1_ref, t_ref, u_ref):
    xb = x_ref[...]
    t_ref[...] = jnp.dot(xb, w1_ref[...],
                         preferred_element_type=jnp.float32) + bm_ref[...]
    u_ref[...] = jnp.dot(xb, wa1_ref[...], preferred_element_type=jnp.float32)


def _pre_tc(x, w1, bm, wa1):
    bn = 1000
    row_block = pl.BlockSpec((bn, D), lambda i: (i, 0))
    full = lambda a, b: pl.BlockSpec((a, b), lambda i: (0, 0))
    return pl.pallas_call(
        _pre_body,
        grid=(N // bn,),
        in_specs=[row_block, full(D, D), full(1, D), full(D, D)],
        out_specs=(row_block, row_block),
        out_shape=(jax.ShapeDtypeStruct((N, D), jnp.float32),
                   jax.ShapeDtypeStruct((N, D), jnp.float32)),
    )(x, w1, bm, wa1)


def _dense_body(t_ref, u_ref, pos_ref, gx_ref, ge_ref, w2_ref, wp_ref,
                wa2_ref, ba_ref, o_ref):
    s = gx_ref[0] + gx_ref[1]
    ge = ge_ref[0, :, :FE] + ge_ref[1, :, :FE]
    sp = ge[:, 0:P]
    counts = ge[:, P:P + 1]
    posb = pos_ref[...]
    q = sp - counts * posb
    wp = wp_ref[...]
    pterm = q[:, 0:1] * wp[0:1, :] + q[:, 1:2] * wp[1:2, :]
    sums = counts * t_ref[...] + pterm + jnp.dot(
        s, w2_ref[...], preferred_element_type=jnp.float32)
    aggr = sums / jnp.maximum(counts, 1.0)
    o_ref[...] = (u_ref[...]
                  + jnp.dot(aggr, wa2_ref[...], preferred_element_type=jnp.float32)
                  + ba_ref[...])


def _dense_tc(t, u, pos, gx, ge, w2, wp, wa2, ba):
    bn = 1000
    grid = (N // bn,)
    row_block = lambda d: pl.BlockSpec((bn, d), lambda i: (i, 0))
    full = lambda a, b: pl.BlockSpec((a, b), lambda i: (0, 0))
    return pl.pallas_call(
        _dense_body,
        grid=grid,
        in_specs=[
            row_block(D), row_block(D), row_block(P),
            pl.BlockSpec((NC, bn, D), lambda i: (0, i, 0)),
            pl.BlockSpec((NC, bn, D), lambda i: (0, i, 0)),
            full(D, D), full(8, D), full(D, D), full(1, D),
        ],
        out_specs=row_block(D),
        out_shape=jax.ShapeDtypeStruct((N, D), jnp.float32),
    )(t, u, pos, gx, ge, w2, wp, wa2, ba)


_PAD_IOTA = np.arange(E_PAD - E, dtype=np.int32)
_PAD_EDGES = np.stack([np.asarray(N + _PAD_IOTA % (N_PAD - N), np.int32),
                       np.asarray(_PAD_IOTA % N, np.int32)])


@jax.jit
def kernel(x, edge_index, pos, W_msg, b_msg, W_aggr, b_aggr):
    xe = jnp.concatenate(
        [pos, jnp.ones((N, 1), jnp.float32),
         jnp.zeros((N, FE - P - 1), jnp.float32)], axis=1)
    xe = jnp.pad(xe, ((0, N_PAD - N), (0, 0)))

    ei = jnp.concatenate([edge_index, jnp.asarray(_PAD_EDGES)], axis=1)
    eis = ei.reshape(2, NW, NB // K, K, B)
    zeros = jnp.zeros((NPT, D), jnp.float32)

    gx, ge = _seg_sum_sc(x, xe, eis, zeros)

    t, u = _pre_tc(x, W_msg[:D], b_msg.reshape(1, D), W_aggr[:D])
    w2 = W_msg[D:2 * D]
    wp = jnp.pad(W_msg[2 * D:], ((0, 8 - P), (0, 0)))
    return _dense_tc(t, u, pos, gx, ge, w2, wp, W_aggr[D:],
                     b_aggr.reshape(1, D))


# speedup vs baseline: 1.0103x; 1.0103x over previous
import functools

import jax
import jax.numpy as jnp
import numpy as np
from jax import lax
from jax.experimental import pallas as pl
from jax.experimental.pallas import tpu as pltpu
from jax.experimental.pallas import tpu_sc as plsc

N = 10000
E = 320000
D = 128
P = 2

NC = 2
NS = 16
NW = NC * NS
F = 144
FE = F - D
B = 128
K = 8
NB = 80
E_PAD = NW * NB * B
N_PAD = 10224
NPT = N_PAD // NS

_sc_mesh = plsc.VectorSubcoreMesh(core_axis_name="c", subcore_axis_name="s")


@functools.partial(
    pl.kernel,
    out_type=(jax.ShapeDtypeStruct((NC, N_PAD, D), jnp.float32),
              jax.ShapeDtypeStruct((NC, N_PAD, D), jnp.float32)),
    mesh=_sc_mesh,
    compiler_params=pltpu.CompilerParams(use_tc_tiling_on_sc=False),
    scratch_types=[
        pltpu.VMEM((K, B), jnp.int32),
        pltpu.VMEM((K, B), jnp.int32),
        pltpu.VMEM((B, D), jnp.float32),
        pltpu.VMEM((B, D), jnp.float32),
        pltpu.VMEM((B, FE), jnp.float32),
        pltpu.VMEM((B, FE), jnp.float32),
        pltpu.VMEM_SHARED((N_PAD, D), jnp.float32),
        pltpu.VMEM_SHARED((N_PAD, FE), jnp.float32),
        pltpu.SemaphoreType.DMA,
        pltpu.SemaphoreType.DMA,
        pltpu.SemaphoreType.DMA,
        pltpu.SemaphoreType.DMA,
    ],
)
def _seg_sum_sc(x_hbm, xe_hbm, edges_hbm, zeros_hbm, outx_hbm,
                oute_hbm, src_v, dst_v, rx0, rx1, re0, re1, accx, acce,
                gsem0, gsem1, ssem0, ssem1):
    cid = lax.axis_index("c")
    sid = lax.axis_index("s")
    wid = cid * NS + sid

    def load_slab(sb):
        pltpu.sync_copy(edges_hbm.at[0, wid, sb], src_v)
        pltpu.sync_copy(edges_hbm.at[1, wid, sb], dst_v)

    def start_gather(k, rx, re, sem):
        pltpu.async_copy(x_hbm.at[dst_v.at[k]], rx, sem)
        pltpu.async_copy(xe_hbm.at[dst_v.at[k]], re, sem)

    def wait_gather(k, rx, re, sem):
        pltpu.make_async_copy(x_hbm.at[dst_v.at[k]], rx, sem).wait()
        pltpu.make_async_copy(xe_hbm.at[dst_v.at[k]], re, sem).wait()

    def start_scatter(k, rx, re, sem):
        pltpu.async_copy(rx, accx.at[src_v.at[k]], sem, add=True)
        pltpu.async_copy(re, acce.at[src_v.at[k]], sem, add=True)

    def wait_scatter(k, rx, re, sem):
        pltpu.make_async_copy(rx, accx.at[src_v.at[k]], sem).wait()
        pltpu.make_async_copy(re, acce.at[src_v.at[k]], sem).wait()

    pltpu.sync_copy(zeros_hbm, accx.at[pl.ds(sid * NPT, NPT)])
    pltpu.sync_copy(zeros_hbm.at[:, pl.ds(0, FE)],
                    acce.at[pl.ds(sid * NPT, NPT)])
    plsc.subcore_barrier()

    def superblock(s, _):
        load_slab(s)
        start_gather(0, rx0, re0, gsem0)
        bufs = [(rx0, re0, gsem0, ssem0), (rx1, re1, gsem1, ssem1)]
        for k in range(K):
            rx, re, gsem, ssem = bufs[k % 2]
            orx, ore, ogsem, ossem = bufs[(k + 1) % 2]
            if k + 1 < K:
                if k >= 1:
                    wait_scatter(k - 1, orx, ore, ossem)
                start_gather(k + 1, orx, ore, ogsem)
            wait_gather(k, rx, re, gsem)
            start_scatter(k, rx, re, ssem)
        wait_scatter(K - 2, rx0, re0, ssem0)
        wait_scatter(K - 1, rx1, re1, ssem1)
        return 0

    lax.fori_loop(0, NB // K, superblock, 0)

    plsc.subcore_barrier()
    r0 = sid * NPT
    pltpu.sync_copy(accx.at[pl.ds(r0, NPT)], outx_hbm.at[cid, pl.ds(r0, NPT)])
    pltpu.sync_copy(acce.at[pl.ds(r0, NPT)],
                    oute_hbm.at[cid, pl.ds(r0, NPT), pl.ds(0, FE)])


def _dense_body(x_ref, pos_ref, gx_ref, ge_ref, w1_ref, w2_ref, wp_ref,
                bm_ref, wa1_ref, wa2_ref, ba_ref, o_ref):
    xb = x_ref[...]
    s = gx_ref[0] + gx_ref[1]
    ge = ge_ref[0, :, :FE] + ge_ref[1, :, :FE]
    sp = ge[:, 0:P]
    counts = ge[:, P:P + 1]
    posb = pos_ref[...]
    q = sp - counts * posb
    wp = wp_ref[...]
    pterm = q[:, 0:1] * wp[0:1, :] + q[:, 1:2] * wp[1:2, :]
    t = jnp.dot(xb, w1_ref[...], preferred_element_type=jnp.float32) + bm_ref[...]
    sums = counts * t + pterm + jnp.dot(s, w2_ref[...],
                                        preferred_element_type=jnp.float32)
    aggr = sums / jnp.maximum(counts, 1.0)
    o_ref[...] = (jnp.dot(xb, wa1_ref[...], preferred_element_type=jnp.float32)
                  + jnp.dot(aggr, wa2_ref[...], preferred_element_type=jnp.float32)
                  + ba_ref[...])


def _dense_tc(x, pos, gx, ge, w1, w2, wp, bm, wa1, wa2, ba):
    bn = 1000
    grid = (N // bn,)
    row_block = lambda d: pl.BlockSpec((bn, d), lambda i: (i, 0))
    full = lambda a, b: pl.BlockSpec((a, b), lambda i: (0, 0))
    return pl.pallas_call(
        _dense_body,
        grid=grid,
        in_specs=[
            row_block(D), row_block(P),
            pl.BlockSpec((NC, bn, D), lambda i: (0, i, 0)),
            pl.BlockSpec((NC, bn, D), lambda i: (0, i, 0)),
            full(D, D), full(D, D), full(8, D), full(1, D),
            full(D, D), full(D, D), full(1, D),
        ],
        out_specs=row_block(D),
        out_shape=jax.ShapeDtypeStruct((N, D), jnp.float32),
    )(x, pos, gx, ge, w1, w2, wp, bm, wa1, wa2, ba)


_PAD_IOTA = np.arange(E_PAD - E, dtype=np.int32)
_PAD_EDGES = np.stack([np.asarray(N + _PAD_IOTA % (N_PAD - N), np.int32),
                       np.asarray(_PAD_IOTA % N, np.int32)])


@jax.jit
def kernel(x, edge_index, pos, W_msg, b_msg, W_aggr, b_aggr):
    xe = jnp.concatenate(
        [pos, jnp.ones((N, 1), jnp.float32),
         jnp.zeros((N, FE - P - 1), jnp.float32)], axis=1)
    xe = jnp.pad(xe, ((0, N_PAD - N), (0, 0)))

    ei = jnp.concatenate([edge_index, jnp.asarray(_PAD_EDGES)], axis=1)
    eis = ei.reshape(2, NW, NB // K, K, B)
    zeros = jnp.zeros((NPT, D), jnp.float32)

    gx, ge = _seg_sum_sc(x, xe, eis, zeros)

    w1 = W_msg[:D]
    w2 = W_msg[D:2 * D]
    wp = jnp.pad(W_msg[2 * D:], ((0, 8 - P), (0, 0)))
    wa1 = W_aggr[:D]
    wa2 = W_aggr[D:]
    return _dense_tc(x, pos, gx, ge, w1, w2, wp,
                     b_msg.reshape(1, D), wa1, wa2, b_aggr.reshape(1, D))
